# Initial kernel scaffold; baseline (speedup 1.0000x reference)
import jax
import jax.numpy as jnp
from jax.experimental import pallas as pl


def kernel(x, neighbors, dists, W, b):
    raise NotImplementedError("write your pallas kernel here")


# speedup vs baseline: 5.2282x; 5.2282x over previous
import functools

import jax
import jax.numpy as jnp
import numpy as np
from jax import lax
from jax.experimental import pallas as pl
from jax.experimental.pallas import tpu as pltpu
from jax.experimental.pallas import tpu_sc as plsc

_B = 2
_T_IN = 12
_T_TOTAL = 24
_N = 10000
_C = 2
_K = 16
_H = 4
_SIGMA = 6.0

_NW = 32
_NPW = 320
_NPAD = _NW * _NPW
_F = _B * _C * _T_IN
_FH = _F * _H


def _sc_aggregate(xr, nbr, dist):
    mesh = plsc.VectorSubcoreMesh(core_axis_name="c", subcore_axis_name="s")

    @functools.partial(
        pl.kernel,
        mesh=mesh,
        compiler_params=pltpu.CompilerParams(use_tc_tiling_on_sc=False),
        out_type=jax.ShapeDtypeStruct((_NPAD, _FH), jnp.float32),
        scratch_types=[
            pltpu.VMEM((_NPW, _K), jnp.int32),
            pltpu.VMEM((_NPW, _K), jnp.float32),
            pltpu.VMEM((2, _K, _F), jnp.float32),
            pltpu.VMEM((_NPW, _FH), jnp.float32),
            pltpu.SemaphoreType.DMA,
            pltpu.SemaphoreType.DMA,
        ],
    )
    def k(xr_hbm, nbr_hbm, dist_hbm, acc_hbm, nbr_v, dist_v, rows_v,
          out_v, gsem0, gsem1):
        wid = lax.axis_index("s") * 2 + lax.axis_index("c")
        base = wid * _NPW
        pltpu.sync_copy(nbr_hbm.at[pl.ds(base, _NPW)], nbr_v)
        pltpu.sync_copy(dist_hbm.at[pl.ds(base, _NPW)], dist_v)

        inv = np.float32(-1.0 / (_SIGMA * _SIGMA * _H))

        def gather_start(i, buf, sem):
            pltpu.async_copy(xr_hbm.at[nbr_v.at[i]], rows_v.at[buf], sem)

        def gather_wait(i, buf, sem):
            pltpu.make_async_copy(xr_hbm.at[nbr_v.at[i]], rows_v.at[buf],
                                  sem).wait()

        def node(i, buf):
            d = dist_v[i]
            u = jnp.exp(d * d * inv)
            acc = [[None] * 3 for _ in range(_H)]
            for kk in range(_K):
                uk = u[kk]
                uk2 = uk * uk
                w = (uk, uk2, uk2 * uk, uk2 * uk2)
                for c in range(3):
                    r = rows_v[buf, kk, pl.ds(c * 16, 16)]
                    for h in range(_H):
                        t = w[h] * r
                        acc[h][c] = t if kk == 0 else acc[h][c] + t
            for h in range(_H):
                for c in range(3):
                    out_v[i, pl.ds(h * _F + c * 16, 16)] = acc[h][c]

        nblocks = _NPW // 2
        gather_start(0, 0, gsem0)

        def body(bi, carry):
            i = bi * 2
            gather_wait(i, 0, gsem0)
            gather_start(i + 1, 1, gsem1)
            node(i, 0)
            gather_wait(i + 1, 1, gsem1)

            @pl.when(bi + 1 < nblocks)
            def _():
                gather_start(i + 2, 0, gsem0)

            node(i + 1, 1)
            return carry

        lax.fori_loop(0, nblocks, body, 0)
        pltpu.sync_copy(out_v, acc_hbm.at[pl.ds(base, _NPW)])

    return k(xr, nbr, dist)


def _tc_shrink(acc, W2, b2):
    blk = _NPAD // 4

    def body(acc_ref, w_ref, b_ref, y_ref):
        y_ref[...] = jnp.maximum(
            jnp.dot(acc_ref[...], w_ref[...],
                    preferred_element_type=jnp.float32) + b_ref[...], 0.0)

    return pl.pallas_call(
        body,
        grid=(4,),
        in_specs=[
            pl.BlockSpec((blk, _FH), lambda i: (i, 0)),
            pl.BlockSpec((_FH, _F), lambda i: (0, 0)),
            pl.BlockSpec((1, _F), lambda i: (0, 0)),
        ],
        out_specs=pl.BlockSpec((blk, _F), lambda i: (i, 0)),
        out_shape=jax.ShapeDtypeStruct((_NPAD, _F), jnp.float32),
    )(acc, W2, b2)


def kernel(x, neighbors, dists, W, b):
    xr = jnp.transpose(x, (2, 0, 3, 1)).reshape(_N, _F)
    nbr = jnp.pad(neighbors.astype(jnp.int32), ((0, _NPAD - _N), (0, 0)))
    dist = jnp.pad(dists, ((0, _NPAD - _N), (0, 0)))
    M = jnp.transpose(W.reshape(_T_IN, _H, _T_TOTAL - _T_IN), (1, 0, 2))
    eye = jnp.eye(_B * _C, dtype=W.dtype)
    W2 = (M[:, None, :, None, :] * eye[None, :, None, :, None]).reshape(_FH, _F)
    b2 = jnp.tile(b, _B * _C).reshape(1, _F)

    acc = _sc_aggregate(xr, nbr, dist)

    y = _tc_shrink(acc, W2, b2)

    yb = y[:_N].reshape(_N, _B, _C, _T_TOTAL - _T_IN)
    yb = jnp.transpose(yb, (1, 3, 0, 2))
    return jnp.concatenate([x, yb], axis=1)


# speedup vs baseline: 6.7198x; 1.2853x over previous
import functools

import jax
import jax.numpy as jnp
import numpy as np
from jax import lax
from jax.experimental import pallas as pl
from jax.experimental.pallas import tpu as pltpu
from jax.experimental.pallas import tpu_sc as plsc

_B = 2
_T_IN = 12
_T_TOTAL = 24
_N = 10000
_C = 2
_K = 16
_H = 4
_SIGMA = 6.0

_NW = 32
_NPW = 320
_NPAD = _NW * _NPW
_F = _B * _C * _T_IN
_FH = _F * _H
_NBUF = 8


def _sc_aggregate(xr, nbr, dist):
    mesh = plsc.VectorSubcoreMesh(core_axis_name="c", subcore_axis_name="s")

    @functools.partial(
        pl.kernel,
        mesh=mesh,
        compiler_params=pltpu.CompilerParams(use_tc_tiling_on_sc=False),
        out_type=jax.ShapeDtypeStruct((_NPAD, _FH), jnp.float32),
        scratch_types=[
            pltpu.VMEM((_NPW + _NBUF, _K), jnp.int32),
            pltpu.VMEM((_NPW, _K), jnp.float32),
            pltpu.VMEM((_NBUF, _K, _F), jnp.float32),
            pltpu.VMEM((_NPW, _FH), jnp.float32),
        ] + [pltpu.SemaphoreType.DMA] * _NBUF,
    )
    def k(xr_hbm, nbr_hbm, dist_hbm, acc_hbm, nbr_v, dist_v, rows_v,
          out_v, *gsems):
        wid = lax.axis_index("s") * 2 + lax.axis_index("c")
        base = wid * _NPW
        pltpu.sync_copy(nbr_hbm.at[pl.ds(base, _NPW)],
                        nbr_v.at[pl.ds(0, _NPW)])
        pltpu.sync_copy(dist_hbm.at[pl.ds(base, _NPW)], dist_v)
        for r in range(_NBUF - 1):
            nbr_v[_NPW + r] = jnp.zeros((_K,), jnp.int32)

        inv = np.float32(-1.0 / (_SIGMA * _SIGMA * _H))

        def gather_start(i, buf):
            pltpu.async_copy(xr_hbm.at[nbr_v.at[i]], rows_v.at[buf],
                             gsems[buf])

        def gather_wait(i, buf):
            pltpu.make_async_copy(xr_hbm.at[nbr_v.at[i]], rows_v.at[buf],
                                  gsems[buf]).wait()

        def node(i, buf):
            d = dist_v[i]
            u = jnp.exp(d * d * inv)
            acc = [[None] * 3 for _ in range(_H)]
            for kk in range(_K):
                uk = u[kk]
                uk2 = uk * uk
                w = (uk, uk2, uk2 * uk, uk2 * uk2)
                for c in range(3):
                    r = rows_v[buf, kk, pl.ds(c * 16, 16)]
                    for h in range(_H):
                        t = w[h] * r
                        acc[h][c] = t if kk == 0 else acc[h][c] + t
            for h in range(_H):
                for c in range(3):
                    out_v[i, pl.ds(h * _F + c * 16, 16)] = acc[h][c]

        for p in range(_NBUF - 1):
            gather_start(p, p)

        def body(bi, carry):
            i0 = bi * _NBUF
            for b in range(_NBUF):
                i = i0 + b
                gather_start(i + _NBUF - 1, (b + _NBUF - 1) % _NBUF)
                gather_wait(i, b)
                node(i, b)
            return carry

        lax.fori_loop(0, _NPW // _NBUF, body, 0)
        for r in range(_NBUF - 1):
            gather_wait(_NPW + r, (_NPW + r) % _NBUF)
        pltpu.sync_copy(out_v, acc_hbm.at[pl.ds(base, _NPW)])

    return k(xr, nbr, dist)


def _tc_shrink(acc, W2, b2):
    blk = _NPAD // 4

    def body(acc_ref, w_ref, b_ref, y_ref):
        y_ref[...] = jnp.maximum(
            jnp.dot(acc_ref[...], w_ref[...],
                    preferred_element_type=jnp.float32) + b_ref[...], 0.0)

    return pl.pallas_call(
        body,
        grid=(4,),
        in_specs=[
            pl.BlockSpec((blk, _FH), lambda i: (i, 0)),
            pl.BlockSpec((_FH, _F), lambda i: (0, 0)),
            pl.BlockSpec((1, _F), lambda i: (0, 0)),
        ],
        out_specs=pl.BlockSpec((blk, _F), lambda i: (i, 0)),
        out_shape=jax.ShapeDtypeStruct((_NPAD, _F), jnp.float32),
    )(acc, W2, b2)


def kernel(x, neighbors, dists, W, b):
    xr = jnp.transpose(x, (2, 0, 3, 1)).reshape(_N, _F)
    nbr = jnp.pad(neighbors.astype(jnp.int32), ((0, _NPAD - _N), (0, 0)))
    dist = jnp.pad(dists, ((0, _NPAD - _N), (0, 0)))
    M = jnp.transpose(W.reshape(_T_IN, _H, _T_TOTAL - _T_IN), (1, 0, 2))
    eye = jnp.eye(_B * _C, dtype=W.dtype)
    W2 = (M[:, None, :, None, :] * eye[None, :, None, :, None]).reshape(_FH, _F)
    b2 = jnp.tile(b, _B * _C).reshape(1, _F)

    acc = _sc_aggregate(xr, nbr, dist)

    y = _tc_shrink(acc, W2, b2)

    yb = y[:_N].reshape(_N, _B, _C, _T_TOTAL - _T_IN)
    yb = jnp.transpose(yb, (1, 3, 0, 2))
    return jnp.concatenate([x, yb], axis=1)
